# Initial kernel scaffold; baseline (speedup 1.0000x reference)
import jax
import jax.numpy as jnp
from jax.experimental import pallas as pl


def kernel(x, gate_W, gate_b, W1, b1, W2, b2, data_task_label):
    raise NotImplementedError("write your pallas kernel here")


# speedup vs baseline: 1.9033x; 1.9033x over previous
import functools

import jax
import jax.numpy as jnp
from jax.experimental import pallas as pl
from jax.experimental.pallas import tpu as pltpu

E = 16
TOPK = 2


def _moe_body(x_ref, gw_ref, gb_ref, w1_ref, b1_ref, w2_ref, b2_ref,
              out_ref, gates_ref):
    e = pl.program_id(0)
    b = x_ref.shape[0]

    @pl.when(e == 0)
    def _():
        gates_ref[...] = (
            jax.lax.dot_general(
                x_ref[...], gw_ref[...], (((1,), (1,)), ((), ())),
                preferred_element_type=jnp.float32)
            + gb_ref[...]
        )

    gates = gates_ref[...]
    lane = jax.lax.broadcasted_iota(jnp.int32, (b, E), 1)
    m1 = jnp.max(gates, axis=1, keepdims=True)
    i1 = jnp.min(jnp.where(gates == m1, lane, E), axis=1, keepdims=True)
    masked = jnp.where(lane == i1, -jnp.inf, gates)
    m2 = jnp.max(masked, axis=1, keepdims=True)
    i2 = jnp.min(jnp.where(masked == m2, lane, E), axis=1, keepdims=True)
    w_top1 = jax.nn.sigmoid(m1 - m2)
    w_top2 = jax.nn.sigmoid(m2 - m1)
    g_col = jnp.where(i1 == e, w_top1, jnp.where(i2 == e, w_top2, 0.0))

    h = jax.lax.dot_general(
        x_ref[...], w1_ref[0], (((1,), (1,)), ((), ())),
        preferred_element_type=jnp.float32) + b1_ref[0]
    h = jnp.maximum(h, 0.0)
    y = jax.lax.dot_general(
        h, w2_ref[0], (((1,), (1,)), ((), ())),
        preferred_element_type=jnp.float32) + b2_ref[0]
    y = jnp.maximum(y, 0.0)

    contrib = g_col * y

    @pl.when(e == 0)
    def _():
        out_ref[...] = contrib

    @pl.when(e > 0)
    def _():
        out_ref[...] += contrib


@functools.partial(jax.jit, static_argnames=())
def kernel(x, gate_W, gate_b, W1, b1, W2, b2, data_task_label):
    task = data_task_label[0]
    gw = jax.lax.dynamic_index_in_dim(gate_W, task, 0, keepdims=False)
    gb = jax.lax.dynamic_index_in_dim(gate_b, task, 0, keepdims=True)
    b, d_in = x.shape
    d_h = W1.shape[1]
    d_out = W2.shape[1]

    out = pl.pallas_call(
        _moe_body,
        grid=(E,),
        in_specs=[
            pl.BlockSpec((b, d_in), lambda e: (0, 0)),
            pl.BlockSpec((E, d_in), lambda e: (0, 0)),
            pl.BlockSpec((1, E), lambda e: (0, 0)),
            pl.BlockSpec((1, d_h, d_in), lambda e: (e, 0, 0)),
            pl.BlockSpec((1, 1, d_h), lambda e: (e, 0, 0)),
            pl.BlockSpec((1, d_out, d_h), lambda e: (e, 0, 0)),
            pl.BlockSpec((1, 1, d_out), lambda e: (e, 0, 0)),
        ],
        out_specs=pl.BlockSpec((b, d_out), lambda e: (0, 0)),
        out_shape=jax.ShapeDtypeStruct((b, d_out), jnp.float32),
        scratch_shapes=[pltpu.VMEM((b, E), jnp.float32)],
        compiler_params=pltpu.CompilerParams(
            dimension_semantics=("arbitrary",),
        ),
    )(x, gw, gb, W1, b1[:, None, :], W2, b2[:, None, :])
    return out


# speedup vs baseline: 1.9042x; 1.0005x over previous
import functools

import jax
import jax.numpy as jnp
from jax.experimental import pallas as pl
from jax.experimental.pallas import tpu as pltpu

E = 16
TOPK = 2


def _moe_body(x_ref, gw_ref, gb_ref, w1_ref, b1_ref, w2_ref, b2_ref,
              out_ref, gates_ref):
    e = pl.program_id(0)
    b = x_ref.shape[0]

    @pl.when(e == 0)
    def _():
        gates_ref[...] = (
            jax.lax.dot_general(
                x_ref[...], gw_ref[...], (((1,), (1,)), ((), ())),
                preferred_element_type=jnp.float32)
            + gb_ref[...]
        )

    gates = gates_ref[...]
    lane = jax.lax.broadcasted_iota(jnp.int32, (b, E), 1)
    m1 = jnp.max(gates, axis=1, keepdims=True)
    i1 = jnp.min(jnp.where(gates == m1, lane, E), axis=1, keepdims=True)
    masked = jnp.where(lane == i1, -jnp.inf, gates)
    m2 = jnp.max(masked, axis=1, keepdims=True)
    i2 = jnp.min(jnp.where(masked == m2, lane, E), axis=1, keepdims=True)
    w_top1 = jax.nn.sigmoid(m1 - m2)
    w_top2 = jax.nn.sigmoid(m2 - m1)
    g_col = jnp.where(i1 == e, w_top1, jnp.where(i2 == e, w_top2, 0.0))

    h = jax.lax.dot_general(
        x_ref[...].astype(jnp.bfloat16), w1_ref[0].astype(jnp.bfloat16),
        (((1,), (1,)), ((), ())),
        preferred_element_type=jnp.float32) + b1_ref[0]
    h = jnp.maximum(h, 0.0)
    y = jax.lax.dot_general(
        h.astype(jnp.bfloat16), w2_ref[0].astype(jnp.bfloat16),
        (((1,), (1,)), ((), ())),
        preferred_element_type=jnp.float32) + b2_ref[0]
    y = jnp.maximum(y, 0.0)

    contrib = g_col * y

    @pl.when(e == 0)
    def _():
        out_ref[...] = contrib

    @pl.when(e > 0)
    def _():
        out_ref[...] += contrib


@functools.partial(jax.jit, static_argnames=())
def kernel(x, gate_W, gate_b, W1, b1, W2, b2, data_task_label):
    task = data_task_label[0]
    gw = jax.lax.dynamic_index_in_dim(gate_W, task, 0, keepdims=False)
    gb = jax.lax.dynamic_index_in_dim(gate_b, task, 0, keepdims=True)
    b, d_in = x.shape
    d_h = W1.shape[1]
    d_out = W2.shape[1]

    out = pl.pallas_call(
        _moe_body,
        grid=(E,),
        in_specs=[
            pl.BlockSpec((b, d_in), lambda e: (0, 0)),
            pl.BlockSpec((E, d_in), lambda e: (0, 0)),
            pl.BlockSpec((1, E), lambda e: (0, 0)),
            pl.BlockSpec((1, d_h, d_in), lambda e: (e, 0, 0)),
            pl.BlockSpec((1, 1, d_h), lambda e: (e, 0, 0)),
            pl.BlockSpec((1, d_out, d_h), lambda e: (e, 0, 0)),
            pl.BlockSpec((1, 1, d_out), lambda e: (e, 0, 0)),
        ],
        out_specs=pl.BlockSpec((b, d_out), lambda e: (0, 0)),
        out_shape=jax.ShapeDtypeStruct((b, d_out), jnp.float32),
        scratch_shapes=[pltpu.VMEM((b, E), jnp.float32)],
        compiler_params=pltpu.CompilerParams(
            dimension_semantics=("arbitrary",),
        ),
    )(x, gw, gb, W1, b1[:, None, :], W2, b2[:, None, :])
    return out


# speedup vs baseline: 2.1572x; 1.1328x over previous
import functools

import jax
import jax.numpy as jnp
from jax.experimental import pallas as pl
from jax.experimental.pallas import tpu as pltpu

E = 16
TOPK = 2


def _moe_body(x_ref, gw_ref, gb_ref, w1_ref, b1_ref, w2_ref, b2_ref,
              out_ref, gates_ref):
    e = pl.program_id(0)
    b = x_ref.shape[0]

    lane = jax.lax.broadcasted_iota(jnp.int32, (b, E), 1)

    @pl.when(e == 0)
    def _():
        gates = (
            jax.lax.dot_general(
                x_ref[...], gw_ref[...], (((1,), (1,)), ((), ())),
                preferred_element_type=jnp.float32)
            + gb_ref[...]
        )
        m1 = jnp.max(gates, axis=1, keepdims=True)
        i1 = jnp.min(jnp.where(gates == m1, lane, E), axis=1, keepdims=True)
        masked = jnp.where(lane == i1, -jnp.inf, gates)
        m2 = jnp.max(masked, axis=1, keepdims=True)
        i2 = jnp.min(jnp.where(masked == m2, lane, E), axis=1, keepdims=True)
        w_top1 = jax.nn.sigmoid(m1 - m2)
        w_top2 = jax.nn.sigmoid(m2 - m1)
        gates_ref[...] = jnp.where(
            lane == i1, w_top1, jnp.where(lane == i2, w_top2, 0.0))

    g_col = jnp.sum(
        jnp.where(lane == e, gates_ref[...], 0.0), axis=1, keepdims=True)

    h = jax.lax.dot_general(
        x_ref[...].astype(jnp.bfloat16), w1_ref[0].astype(jnp.bfloat16),
        (((1,), (1,)), ((), ())),
        preferred_element_type=jnp.float32) + b1_ref[0]
    h = jnp.maximum(h, 0.0)
    y = jax.lax.dot_general(
        h.astype(jnp.bfloat16), w2_ref[0].astype(jnp.bfloat16),
        (((1,), (1,)), ((), ())),
        preferred_element_type=jnp.float32) + b2_ref[0]
    y = jnp.maximum(y, 0.0)

    contrib = g_col * y

    @pl.when(e == 0)
    def _():
        out_ref[...] = contrib

    @pl.when(e > 0)
    def _():
        out_ref[...] += contrib


@functools.partial(jax.jit, static_argnames=())
def kernel(x, gate_W, gate_b, W1, b1, W2, b2, data_task_label):
    task = data_task_label[0]
    gw = jax.lax.dynamic_index_in_dim(gate_W, task, 0, keepdims=False)
    gb = jax.lax.dynamic_index_in_dim(gate_b, task, 0, keepdims=True)
    b, d_in = x.shape
    d_h = W1.shape[1]
    d_out = W2.shape[1]

    out = pl.pallas_call(
        _moe_body,
        grid=(E,),
        in_specs=[
            pl.BlockSpec((b, d_in), lambda e: (0, 0)),
            pl.BlockSpec((E, d_in), lambda e: (0, 0)),
            pl.BlockSpec((1, E), lambda e: (0, 0)),
            pl.BlockSpec((1, d_h, d_in), lambda e: (e, 0, 0)),
            pl.BlockSpec((1, 1, d_h), lambda e: (e, 0, 0)),
            pl.BlockSpec((1, d_out, d_h), lambda e: (e, 0, 0)),
            pl.BlockSpec((1, 1, d_out), lambda e: (e, 0, 0)),
        ],
        out_specs=pl.BlockSpec((b, d_out), lambda e: (0, 0)),
        out_shape=jax.ShapeDtypeStruct((b, d_out), jnp.float32),
        scratch_shapes=[pltpu.VMEM((b, E), jnp.float32)],
        compiler_params=pltpu.CompilerParams(
            dimension_semantics=("arbitrary",),
        ),
    )(x, gw, gb, W1, b1[:, None, :], W2, b2[:, None, :])
    return out
